# Initial kernel scaffold; baseline (speedup 1.0000x reference)
import jax
import jax.numpy as jnp
from jax.experimental import pallas as pl


def kernel(x, edge_index, batch_idx, W1, b1, W2, b2, W3, b3, lin1_W, lin1_b, lin2_W, lin2_b):
    raise NotImplementedError("write your pallas kernel here")


# speedup vs baseline: 6.6047x; 6.6047x over previous
import functools

import jax
import jax.numpy as jnp
from jax import lax
from jax.experimental import pallas as pl
from jax.experimental.pallas import tpu as pltpu
from jax.experimental.pallas import tpu_sc as plsc

N = 10000
E = 320000
D = 128
H = 128
G = 512

NC = 2
NS = 16
NW = NC * NS

NPAD = 10240
EPAD = 327680
GPAD = 640

EK = 128
ESTEPS = (EPAD // NW) // EK
PK = 64
PSTEPS = (NPAD // NW) // PK
NROWS_T = NPAD // NS
GROWS_T = GPAD // NS

_f32 = jnp.float32
_i32 = jnp.int32


def _mesh():
    return plsc.VectorSubcoreMesh(core_axis_name="c", subcore_axis_name="s",
                                  num_cores=NC, num_subcores=NS)


def _degcnt_body(dst3, bat3, z128, ones_hbm, deg_out, cnt_out,
                 dst_v, bat_v, ones_v, deg_sp, cnt_sp):
    c = lax.axis_index("c")
    s = lax.axis_index("s")
    wid = s * NC + c
    pltpu.sync_copy(dst3.at[wid], dst_v)
    pltpu.sync_copy(bat3.at[wid], bat_v)
    pltpu.sync_copy(ones_hbm, ones_v)
    pltpu.sync_copy(z128, deg_sp.at[pl.ds(s * NROWS_T, NROWS_T)])
    pltpu.sync_copy(z128.at[pl.ds(0, GROWS_T)], cnt_sp.at[pl.ds(s * GROWS_T, GROWS_T)])
    plsc.subcore_barrier()

    def estep(i, carry):
        pltpu.sync_copy(ones_v, deg_sp.at[dst_v.at[i]], add=True)
        return carry
    lax.fori_loop(0, ESTEPS, estep, 0)

    def pstep(j, carry):
        pltpu.sync_copy(ones_v.at[pl.ds(0, PK)], cnt_sp.at[bat_v.at[j]], add=True)
        return carry
    lax.fori_loop(0, PSTEPS, pstep, 0)

    plsc.subcore_barrier()
    pltpu.sync_copy(deg_sp.at[pl.ds(s * NROWS_T, NROWS_T)],
                    deg_out.at[c].at[pl.ds(s * NROWS_T, NROWS_T)])
    pltpu.sync_copy(cnt_sp.at[pl.ds(s * GROWS_T, GROWS_T)],
                    cnt_out.at[c].at[pl.ds(s * GROWS_T, GROWS_T)])


def _make_degcnt():
    return functools.partial(
        pl.kernel,
        out_type=[jax.ShapeDtypeStruct((NC, NPAD, D), _f32),
                  jax.ShapeDtypeStruct((NC, GPAD, D), _f32)],
        mesh=_mesh(),
        scratch_types=[
            pltpu.VMEM((ESTEPS, EK), _i32),
            pltpu.VMEM((PSTEPS, PK), _i32),
            pltpu.VMEM((EK, D), _f32),
            pltpu.VMEM_SHARED((NPAD, D), _f32),
            pltpu.VMEM_SHARED((GPAD, D), _f32),
        ],
    )(_degcnt_body)


def _edge_body(g_hbm, src3, dst3, z128, out_hbm,
               src_v, dst_v, gbuf, acc_sp, sem):
    c = lax.axis_index("c")
    s = lax.axis_index("s")
    wid = s * NC + c
    pltpu.sync_copy(src3.at[wid], src_v)
    pltpu.sync_copy(dst3.at[wid], dst_v)
    pltpu.sync_copy(z128, acc_sp.at[pl.ds(s * NROWS_T, NROWS_T)])
    plsc.subcore_barrier()

    def step(i, carry):
        pltpu.async_copy(g_hbm.at[src_v.at[i]], gbuf, sem).wait()
        pltpu.sync_copy(gbuf, acc_sp.at[dst_v.at[i]], add=True)
        return carry
    lax.fori_loop(0, ESTEPS, step, 0)

    plsc.subcore_barrier()
    pltpu.sync_copy(acc_sp.at[pl.ds(s * NROWS_T, NROWS_T)],
                    out_hbm.at[c].at[pl.ds(s * NROWS_T, NROWS_T)])


def _make_edge():
    return functools.partial(
        pl.kernel,
        out_type=jax.ShapeDtypeStruct((NC, NPAD, D), _f32),
        mesh=_mesh(),
        scratch_types=[
            pltpu.VMEM((ESTEPS, EK), _i32),
            pltpu.VMEM((ESTEPS, EK), _i32),
            pltpu.VMEM((EK, D), _f32),
            pltpu.VMEM_SHARED((NPAD, D), _f32),
            pltpu.SemaphoreType.DMA,
        ],
    )(_edge_body)


def _pool_body(h_hbm, bat3, z128, out_hbm, bat_v, hbuf, pool_sp):
    c = lax.axis_index("c")
    s = lax.axis_index("s")
    wid = s * NC + c
    pltpu.sync_copy(bat3.at[wid], bat_v)
    pltpu.sync_copy(z128.at[pl.ds(0, GROWS_T)], pool_sp.at[pl.ds(s * GROWS_T, GROWS_T)])
    plsc.subcore_barrier()

    def step(j, carry):
        pltpu.sync_copy(h_hbm.at[pl.ds(wid * (NPAD // NW) + j * PK, PK)], hbuf)
        pltpu.sync_copy(hbuf, pool_sp.at[bat_v.at[j]], add=True)
        return carry
    lax.fori_loop(0, PSTEPS, step, 0)

    plsc.subcore_barrier()
    pltpu.sync_copy(pool_sp.at[pl.ds(s * GROWS_T, GROWS_T)],
                    out_hbm.at[c].at[pl.ds(s * GROWS_T, GROWS_T)])


def _make_pool():
    return functools.partial(
        pl.kernel,
        out_type=jax.ShapeDtypeStruct((NC, GPAD, D), _f32),
        mesh=_mesh(),
        scratch_types=[
            pltpu.VMEM((PSTEPS, PK), _i32),
            pltpu.VMEM((PK, D), _f32),
            pltpu.VMEM_SHARED((GPAD, D), _f32),
        ],
    )(_pool_body)


_BLK = 1024
_GRID = NPAD // _BLK


def _dis_inv(degp):
    deg = degp[0, :, 0:1] + degp[1, :, 0:1] + 1.0
    return lax.rsqrt(deg), 1.0 / deg


def _tc_first_body(x_ref, W_ref, degp_ref, h_ref, g_ref):
    dis, _ = _dis_inv(degp_ref[...])
    h = jnp.dot(x_ref[...], W_ref[...], preferred_element_type=_f32)
    h_ref[...] = h
    g_ref[...] = h * dis


def _tc_mid_body(acc_ref, hprev_ref, b_ref, W_ref, degp_ref, h_ref, g_ref):
    acc = acc_ref[...]
    dis, invdeg = _dis_inv(degp_ref[...])
    a = jax.nn.relu(dis * (acc[0] + acc[1]) + hprev_ref[...] * invdeg + b_ref[...])
    h = jnp.dot(a, W_ref[...], preferred_element_type=_f32)
    h_ref[...] = h
    g_ref[...] = h * dis


def _tc_last_body(acc_ref, hprev_ref, b_ref, degp_ref, hp_ref):
    acc = acc_ref[...]
    dis, invdeg = _dis_inv(degp_ref[...])
    hp_ref[...] = jax.nn.relu(dis * (acc[0] + acc[1])
                              + hprev_ref[...] * invdeg + b_ref[...])


def _node_spec():
    return pl.BlockSpec((_BLK, D), lambda i: (i, 0))


def _degp_spec():
    return pl.BlockSpec((NC, _BLK, D), lambda i: (0, i, 0))


def _full_spec(shape):
    nd = len(shape)
    return pl.BlockSpec(shape, lambda i: (0,) * nd)


def _tc_first(x, W, degp):
    return pl.pallas_call(
        _tc_first_body,
        grid=(_GRID,),
        in_specs=[_node_spec(), _full_spec((D, H)), _degp_spec()],
        out_specs=[_node_spec(), _node_spec()],
        out_shape=[jax.ShapeDtypeStruct((NPAD, H), _f32),
                   jax.ShapeDtypeStruct((NPAD, H), _f32)],
    )(x, W, degp)


def _tc_mid(acc, hprev, b, W, degp):
    return pl.pallas_call(
        _tc_mid_body,
        grid=(_GRID,),
        in_specs=[pl.BlockSpec((NC, _BLK, H), lambda i: (0, i, 0)),
                  _node_spec(), _full_spec((1, H)), _full_spec((H, H)),
                  _degp_spec()],
        out_specs=[_node_spec(), _node_spec()],
        out_shape=[jax.ShapeDtypeStruct((NPAD, H), _f32),
                   jax.ShapeDtypeStruct((NPAD, H), _f32)],
    )(acc, hprev, b, W, degp)


def _tc_last(acc, hprev, b, degp):
    return pl.pallas_call(
        _tc_last_body,
        grid=(_GRID,),
        in_specs=[pl.BlockSpec((NC, _BLK, H), lambda i: (0, i, 0)),
                  _node_spec(), _full_spec((1, H)), _degp_spec()],
        out_specs=_node_spec(),
        out_shape=jax.ShapeDtypeStruct((NPAD, H), _f32),
    )(acc, hprev, b, degp)


def _tc_head_body(pool_ref, cnt_ref, W1_ref, b1_ref, W2_ref, b2_ref, out_ref):
    cnt = cnt_ref[0, :, 0:1] + cnt_ref[1, :, 0:1]
    pooled = (pool_ref[0] + pool_ref[1]) / jnp.maximum(cnt, 1.0)
    u = jax.nn.relu(jnp.dot(pooled, W1_ref[...], preferred_element_type=_f32)
                    + b1_ref[...])
    out_ref[...] = jnp.dot(u, W2_ref[...], preferred_element_type=_f32) + b2_ref[...]


def _tc_head(pool, cnt, lin1_W, lin1_b, lin2_Wp, lin2_bp):
    return pl.pallas_call(
        _tc_head_body,
        grid=(1,),
        in_specs=[pl.BlockSpec((NC, G, H), lambda i: (0, 0, 0)),
                  pl.BlockSpec((NC, G, D), lambda i: (0, 0, 0)),
                  _full_spec((H, H)), _full_spec((1, H)),
                  _full_spec((H, H)), _full_spec((1, H))],
        out_specs=pl.BlockSpec((G, H), lambda i: (0, 0)),
        out_shape=jax.ShapeDtypeStruct((G, H), _f32),
    )(pool, cnt, lin1_W, lin1_b, lin2_Wp, lin2_bp)


def kernel(x, edge_index, batch_idx, W1, b1, W2, b2, W3, b3,
           lin1_W, lin1_b, lin2_W, lin2_b):
    src = edge_index[0]
    dst = edge_index[1]

    src3 = jnp.pad(src, (0, EPAD - E)).reshape(NW, ESTEPS, EK)
    dst3 = jnp.pad(dst, (0, EPAD - E), constant_values=N).reshape(NW, ESTEPS, EK)
    bat3 = jnp.pad(batch_idx, (0, NPAD - N), constant_values=G).reshape(NW, PSTEPS, PK)
    xp = jnp.pad(x, ((0, NPAD - N), (0, 0)))
    z128 = jnp.zeros((NROWS_T, D), _f32)
    ones128 = jnp.ones((EK, D), _f32)
    b1r = b1.reshape(1, H)
    b2r = b2.reshape(1, H)
    b3r = b3.reshape(1, H)
    lin1_br = lin1_b.reshape(1, H)
    lin2_Wp = jnp.pad(lin2_W, ((0, 0), (0, H - 1)))
    lin2_bp = jnp.pad(lin2_b, (0, H - 1)).reshape(1, H)

    degp, cntp = _make_degcnt()(dst3, bat3, z128, ones128)

    edge = _make_edge()
    h1, g1 = _tc_first(xp, W1, degp)
    acc1 = edge(g1, src3, dst3, z128)
    h2, g2 = _tc_mid(acc1, h1, b1r, W2, degp)
    acc2 = edge(g2, src3, dst3, z128)
    h3, g3 = _tc_mid(acc2, h2, b2r, W3, degp)
    acc3 = edge(g3, src3, dst3, z128)
    hp = _tc_last(acc3, h3, b3r, degp)

    pool = _make_pool()(hp, bat3, z128)
    out = _tc_head(pool, cntp, lin1_W, lin1_br, lin2_Wp, lin2_bp)
    return out[:, 0:1]
